# baseline (speedup 1.0000x reference)
import functools

import jax
import jax.numpy as jnp
from jax import lax
from jax.experimental import pallas as pl
from jax.experimental.pallas import tpu as pltpu
from jax.experimental.pallas import tpu_sc as plsc

LANES = 16
TRASH = 64


def _eproj_call(edge_attr, Weh, E, DE, HALF):
    BE = 2048

    def body(ea_ref, we_ref, out_ref):
        out_ref[0] = jnp.dot(ea_ref[...], we_ref[0],
                             precision=lax.Precision.HIGHEST,
                             preferred_element_type=jnp.float32)

    return pl.pallas_call(
        body,
        grid=(2, E // BE),
        in_specs=[
            pl.BlockSpec((BE, DE), lambda h, i: (i, 0)),
            pl.BlockSpec((1, DE, HALF), lambda h, i: (h, 0, 0)),
        ],
        out_specs=pl.BlockSpec((1, BE, HALF), lambda h, i: (h, i, 0)),
        out_shape=jax.ShapeDtypeStruct((2, E, HALF), jnp.float32),
    )(edge_attr, Weh)


def _sc_agg_call(x2, ep2, src, dst, N, E, HALF, ph):
    NS = 16
    EperS = E // NS
    K = 64
    NCHUNK = EperS // K
    ZK = 40
    N2 = N // 2

    src3d = src.reshape(NS * NCHUNK, K)
    dst3d = dst.reshape(NS * NCHUNK, K)

    mesh = plsc.VectorSubcoreMesh(core_axis_name="c", subcore_axis_name="s")

    @functools.partial(
        pl.kernel,
        out_type=jax.ShapeDtypeStruct((N, HALF), jnp.float32),
        mesh=mesh,
        scratch_types=[
            pltpu.VMEM((NCHUNK, K), jnp.int32),
            pltpu.VMEM((NCHUNK, K), jnp.int32),
            pltpu.VMEM((2, K, HALF), jnp.float32),
            pltpu.VMEM((2, K, HALF), jnp.float32),
            pltpu.VMEM((2, K, HALF), jnp.float32),
            pltpu.VMEM_SHARED((N2 + TRASH, HALF), jnp.float32),
            pltpu.SemaphoreType.DMA,
            pltpu.SemaphoreType.DMA,
            pltpu.SemaphoreType.DMA,
            pltpu.SemaphoreType.DMA,
            pltpu.SemaphoreType.DMA,
            pltpu.SemaphoreType.DMA,
        ],
    )
    def sc_agg(x2_hbm, ep_hbm, src_hbm, dst_hbm, out_hbm,
               srcv, dstv, xbuf, ebuf, mbuf, agg_sh,
               gsem0, gsem1, esem0, esem1, ssem0, ssem1):
        c = lax.axis_index("c")
        s = lax.axis_index("s")
        ebase = s * EperS
        gsems = (gsem0, gsem1)
        esems = (esem0, esem1)
        ssems = (ssem0, ssem1)

        pltpu.sync_copy(src_hbm.at[pl.ds(s * NCHUNK, NCHUNK), :], srcv)
        pltpu.sync_copy(dst_hbm.at[pl.ds(s * NCHUNK, NCHUNK), :], dstv)

        def shift_body(r, carry):
            for j in range(K // LANES):
                sl = pl.ds(j * LANES, LANES)
                srcv[r, sl] = srcv[r, sl] * 2 + c
                d = dstv[r, sl] - ph * N2
                inr = (d >= 0) & (d < N2)
                dstv[r, sl] = jnp.where(inr, d,
                                        N2 + (dstv[r, sl] & (TRASH - 1)))
            return carry

        lax.fori_loop(0, NCHUNK, shift_body, 0, unroll=2)

        def issue_fetch(i, b):
            pltpu.async_copy(x2_hbm.at[srcv.at[i]],
                             xbuf.at[b], gsems[b])
            pltpu.async_copy(ep_hbm.at[pl.ds(c * E + ebase + i * K, K)],
                             ebuf.at[b], esems[b])

        issue_fetch(0, 0)
        issue_fetch(1, 1)

        def zrow(r, carry):
            for j in range(HALF // LANES):
                mbuf[0, r, pl.ds(j * LANES, LANES)] = jnp.zeros((LANES,),
                                                                jnp.float32)
            return carry

        lax.fori_loop(0, ZK, zrow, 0)
        nchunk_a = (N2 + TRASH) // ZK
        for t in range(-(-nchunk_a // NS)):
            idx = s + NS * t

            @pl.when(idx < nchunk_a)
            def _():
                pltpu.sync_copy(mbuf.at[0, pl.ds(0, ZK)],
                                agg_sh.at[pl.ds(idx * ZK, ZK)])

        plsc.subcore_barrier()

        def process(i, b):
            pltpu.make_async_copy(x2_hbm.at[pl.ds(0, K)], xbuf.at[b],
                                  gsems[b]).wait()
            pltpu.make_async_copy(ep_hbm.at[pl.ds(0, K)], ebuf.at[b],
                                  esems[b]).wait()

            @pl.when(i >= 2)
            def _():
                pltpu.make_async_copy(x2_hbm.at[pl.ds(0, K)],
                                      agg_sh.at[pl.ds(0, K)],
                                      ssems[b]).wait()

            def row(r, rcarry):
                for j in range(HALF // LANES):
                    sl = pl.ds(j * LANES, LANES)
                    mbuf[b, r, sl] = jnp.maximum(
                        xbuf[b, r, sl] + ebuf[b, r, sl], 0.0)
                return rcarry

            lax.fori_loop(0, K, row, 0, unroll=2)

            pltpu.async_copy(mbuf.at[b], agg_sh.at[dstv.at[i]],
                             ssems[b], add=True)

            @pl.when(i + 2 < NCHUNK)
            def _():
                issue_fetch(i + 2, b)

        def pair(p, carry):
            process(2 * p, 0)
            process(2 * p + 1, 1)
            return carry

        lax.fori_loop(0, NCHUNK // 2, pair, 0)
        if NCHUNK % 2:
            process(NCHUNK - 1, 0)

        last = NCHUNK - 1
        pltpu.make_async_copy(x2_hbm.at[pl.ds(0, K)], agg_sh.at[pl.ds(0, K)],
                              ssems[last % 2]).wait()
        pltpu.make_async_copy(x2_hbm.at[pl.ds(0, K)], agg_sh.at[pl.ds(0, K)],
                              ssems[(last - 1) % 2]).wait()
        plsc.subcore_barrier()

        nchunk_n = N2 // ZK
        for t in range(-(-nchunk_n // NS)):
            idx = s + NS * t

            @pl.when(idx < nchunk_n)
            def _():
                pltpu.sync_copy(agg_sh.at[pl.ds(idx * ZK, ZK)],
                                out_hbm.at[pl.ds(c * N2 + idx * ZK, ZK)])

    return sc_agg(x2, ep2, src3d, dst3d)


def _mlp_call(x, agg, Wc, bc2, W1, b12, W2, b22, N, DF, HALF, DC, DH):
    BN = 1000
    nb = N // BN
    hb = nb // 2

    def body(x_ref, g0_ref, g1_ref, wc_ref, bc_ref, w1_ref, b1_ref,
             w2_ref, b2_ref, out_ref):
        hi = lax.Precision.HIGHEST
        a0 = x_ref[:, :HALF] + g0_ref[...]
        a1 = x_ref[:, HALF:] + g1_ref[...]
        h = (jnp.dot(a0, wc_ref[:HALF, :], precision=hi,
                     preferred_element_type=jnp.float32)
             + jnp.dot(a1, wc_ref[HALF:, :], precision=hi,
                       preferred_element_type=jnp.float32)
             + bc_ref[...])
        h = jnp.maximum(
            jnp.dot(h, w1_ref[...], precision=hi,
                    preferred_element_type=jnp.float32)
            + b1_ref[...], 0.0)
        out_ref[...] = (jnp.dot(h, w2_ref[...], precision=hi,
                                preferred_element_type=jnp.float32)
                        + b2_ref[...])

    return pl.pallas_call(
        body,
        grid=(nb,),
        in_specs=[
            pl.BlockSpec((BN, DF), lambda i: (i, 0)),
            pl.BlockSpec((BN, HALF), lambda i, _hb=hb: (i + (i // _hb) * _hb,
                                                        0)),
            pl.BlockSpec((BN, HALF),
                         lambda i, _hb=hb: (i + (i // _hb) * _hb + _hb, 0)),
            pl.BlockSpec((DF, DC), lambda i: (0, 0)),
            pl.BlockSpec((1, DC), lambda i: (0, 0)),
            pl.BlockSpec((DC, DH), lambda i: (0, 0)),
            pl.BlockSpec((1, DH), lambda i: (0, 0)),
            pl.BlockSpec((DH, DF), lambda i: (0, 0)),
            pl.BlockSpec((1, DF), lambda i: (0, 0)),
        ],
        out_specs=pl.BlockSpec((BN, DF), lambda i: (i, 0)),
        out_shape=jax.ShapeDtypeStruct((N, DF), jnp.float32),
    )(x, agg, agg, Wc, bc2, W1, b12, W2, b22)


def kernel(x, edge_index, edge_attr, We, Wc, bc, W1, b1, W2, b2):
    N, DF = x.shape
    E = edge_index.shape[1]
    DE = edge_attr.shape[1]
    DC = Wc.shape[1]
    DH = W1.shape[1]
    HALF = DF // 2

    EP = 16 * 128 * 80
    pad = EP - E
    src = jnp.concatenate([edge_index[0], jnp.zeros((pad,), jnp.int32)])
    dst = jnp.concatenate([edge_index[1], jnp.full((pad,), -1, jnp.int32)])
    ea_p = jnp.concatenate(
        [edge_attr, jnp.zeros((pad, DE), jnp.float32)], axis=0)

    x2 = x.reshape(2 * N, HALF)
    Weh = We.reshape(DE, 2, HALF).transpose(1, 0, 2)

    ep = _eproj_call(ea_p, Weh, EP, DE, HALF)
    ep2 = ep.reshape(2 * EP, HALF)

    aggA = _sc_agg_call(x2, ep2, src, dst, N, EP, HALF, 0)
    aggB = _sc_agg_call(x2, ep2, src, dst, N, EP, HALF, 1)
    agg = jnp.concatenate([aggA, aggB], axis=0)

    out = _mlp_call(x, agg, Wc, bc.reshape(1, DC), W1, b1.reshape(1, DH),
                    W2, b2.reshape(1, DF), N, DF, HALF, DC, DH)
    return out
